# baseline (speedup 1.0000x reference)
import functools

import jax
import jax.numpy as jnp
from jax import lax
from jax.experimental import pallas as pl
from jax.experimental.pallas import tpu as pltpu
from jax.experimental.pallas import tpu_sc as plsc

NC = 2
NS = 16
NW = NC * NS
CHUNK = 128
NBUF = 7


@functools.partial(jax.jit, static_argnames=("b_total", "d"))
def _sc_gather(ts_flat, pe, b_total, d):
    b_per_w = b_total // NW
    n_chunks = b_per_w // CHUNK
    rounds = n_chunks // NBUF
    mesh = plsc.VectorSubcoreMesh(core_axis_name="c", subcore_axis_name="s")

    @functools.partial(
        pl.kernel,
        mesh=mesh,
        out_type=jax.ShapeDtypeStruct((b_total, d), jnp.float32),
        scratch_types=[
            pltpu.VMEM((b_per_w,), jnp.int32),
            pltpu.VMEM((NBUF, CHUNK, d), jnp.float32),
        ]
        + [pltpu.SemaphoreType.DMA] * (2 * NBUF),
    )
    def k(idx_hbm, table_hbm, out_hbm, idx_v, rows_v, *sems):
        gsem = sems[:NBUF]
        wsem = sems[NBUF:]
        sid = lax.axis_index("s")
        wid = sid * NC + lax.axis_index("c")
        base = wid * b_per_w

        pltpu.sync_copy(idx_hbm.at[pl.ds(base, b_per_w)], idx_v)

        def start_gather(g, b):
            pltpu.async_copy(
                table_hbm.at[idx_v.at[pl.ds(g * CHUNK, CHUNK)]],
                rows_v.at[b],
                gsem[b],
            )

        def wait_gather(g, b):
            pltpu.make_async_copy(
                table_hbm.at[idx_v.at[pl.ds(g * CHUNK, CHUNK)]],
                rows_v.at[b],
                gsem[b],
            ).wait()

        def start_write(g, b):
            pltpu.async_copy(
                rows_v.at[b],
                out_hbm.at[pl.ds(base + g * CHUNK, CHUNK)],
                wsem[b],
            )

        def wait_write(g, b):
            pltpu.make_async_copy(
                rows_v.at[b],
                out_hbm.at[pl.ds(base + g * CHUNK, CHUNK)],
                wsem[b],
            ).wait()

        for b in range(NBUF):
            start_gather(b, b)

        def body(r, carry):
            g0 = r * NBUF
            for b in range(NBUF):
                wait_gather(g0 + b, b)
                start_write(g0 + b, b)
            for b in range(NBUF):
                wait_write(g0 + b, b)
                start_gather(g0 + NBUF + b, b)
            return carry

        lax.fori_loop(0, rounds - 1, body, 0)

        left = n_chunks - rounds * NBUF
        g0 = (rounds - 1) * NBUF
        for b in range(NBUF):
            wait_gather(g0 + b, b)
            start_write(g0 + b, b)
        for b in range(NBUF):
            wait_write(g0 + b, b)
            if b < left:
                start_gather(rounds * NBUF + b, b)
        for b in range(left):
            wait_gather(rounds * NBUF + b, b)
            start_write(rounds * NBUF + b, b)
        for b in range(left):
            wait_write(rounds * NBUF + b, b)

    return k(ts_flat, pe)


def kernel(timestamps, pe):
    b, h = timestamps.shape
    d = pe.shape[1]
    hp = (h + 7) // 8 * 8
    idx2 = jnp.pad(timestamps, ((0, 0), (0, hp - h)))
    out = _sc_gather(idx2.reshape(-1), pe, b * hp, d)
    return out.reshape(b, hp, d)[:, :h, :]


# speedup vs baseline: 6.1901x; 6.1901x over previous
import functools

import jax
import jax.numpy as jnp
from jax import lax
from jax.experimental import pallas as pl
from jax.experimental.pallas import tpu as pltpu
from jax.experimental.pallas import tpu_sc as plsc

NC = 2
NS = 16
NW = NC * NS
CHUNK = 128
NBUF = 7


@functools.partial(jax.jit, static_argnames=("b_total", "d"))
def _sc_gather(ts_flat, pe, b_total, d):
    b_per_w = b_total // NW
    n_chunks = b_per_w // CHUNK
    rounds = n_chunks // NBUF
    mesh = plsc.VectorSubcoreMesh(core_axis_name="c", subcore_axis_name="s")

    @functools.partial(
        pl.kernel,
        mesh=mesh,
        out_type=jax.ShapeDtypeStruct((b_total, d), jnp.float32),
        scratch_types=[
            pltpu.VMEM((b_per_w,), jnp.int32),
            pltpu.VMEM((NBUF, CHUNK, d), jnp.float32),
        ]
        + [pltpu.SemaphoreType.DMA] * (2 * NBUF),
    )
    def k(idx_hbm, table_hbm, out_hbm, idx_v, rows_v, *sems):
        gsem = sems[:NBUF]
        wsem = sems[NBUF:]
        sid = lax.axis_index("s")
        wid = sid * NC + lax.axis_index("c")
        base = wid * b_per_w

        pltpu.sync_copy(idx_hbm.at[pl.ds(base, b_per_w)], idx_v)

        def start_gather(g, b):
            pltpu.async_copy(
                table_hbm.at[idx_v.at[pl.ds(g * CHUNK, CHUNK)]],
                rows_v.at[b],
                gsem[b],
            )

        def wait_gather(g, b):
            pltpu.make_async_copy(
                table_hbm.at[idx_v.at[pl.ds(g * CHUNK, CHUNK)]],
                rows_v.at[b],
                gsem[b],
            ).wait()

        def start_write(g, b):
            pltpu.async_copy(
                rows_v.at[b],
                out_hbm.at[pl.ds(base + g * CHUNK, CHUNK)],
                wsem[b],
            )

        def wait_write(g, b):
            pltpu.make_async_copy(
                rows_v.at[b],
                out_hbm.at[pl.ds(base + g * CHUNK, CHUNK)],
                wsem[b],
            ).wait()

        for b in range(NBUF):
            start_gather(b, b)

        def body(r, carry):
            g0 = r * NBUF
            for b in range(NBUF):
                wait_gather(g0 + b, b)
                start_write(g0 + b, b)
            for b in range(NBUF):
                wait_write(g0 + b, b)
                start_gather(g0 + NBUF + b, b)
            return carry

        lax.fori_loop(0, rounds - 1, body, 0)

        left = n_chunks - rounds * NBUF
        g0 = (rounds - 1) * NBUF
        for b in range(NBUF):
            wait_gather(g0 + b, b)
            start_write(g0 + b, b)
        for b in range(NBUF):
            wait_write(g0 + b, b)
            if b < left:
                start_gather(rounds * NBUF + b, b)
        for b in range(left):
            wait_gather(rounds * NBUF + b, b)
            start_write(rounds * NBUF + b, b)
        for b in range(left):
            wait_write(rounds * NBUF + b, b)

    return k(ts_flat, pe)


def kernel(timestamps, pe):
    b, h = timestamps.shape
    d = pe.shape[1]
    hp = (h + 7) // 8 * 8
    idx2 = jnp.pad(timestamps, ((0, 0), (0, hp - h)), mode="edge")
    out = _sc_gather(idx2.reshape(-1), pe, b * hp, d)
    return out.reshape(b, hp, d)[:, :h, :]


# speedup vs baseline: 7.6187x; 1.2308x over previous
import functools

import jax
import jax.numpy as jnp
from jax import lax
from jax.experimental import pallas as pl
from jax.experimental.pallas import tpu as pltpu
from jax.experimental.pallas import tpu_sc as plsc

NC = 2
NS = 16
NW = NC * NS
NBUF = 8


@functools.partial(jax.jit, static_argnames=("b", "h", "hp", "d"))
def _sc_gather(idx2_flat, pe, b, h, hp, d):
    batches_per_w = b // NW
    rounds = batches_per_w // NBUF
    idx_per_w = batches_per_w * hp
    mesh = plsc.VectorSubcoreMesh(core_axis_name="c", subcore_axis_name="s")

    @functools.partial(
        pl.kernel,
        mesh=mesh,
        out_type=jax.ShapeDtypeStruct((b, h, d), jnp.float32),
        scratch_types=[
            pltpu.VMEM((idx_per_w,), jnp.int32),
            pltpu.VMEM((NBUF, h, d), jnp.float32),
        ]
        + [pltpu.SemaphoreType.DMA] * (2 * NBUF),
    )
    def k(idx_hbm, table_hbm, out_hbm, idx_v, rows_v, *sems):
        gsem = sems[:NBUF]
        wsem = sems[NBUF:]
        wid = lax.axis_index("s") * NC + lax.axis_index("c")
        batch0 = wid * batches_per_w

        pltpu.sync_copy(idx_hbm.at[pl.ds(wid * idx_per_w, idx_per_w)], idx_v)

        def start_gather(j, bf):
            pltpu.async_copy(
                table_hbm.at[idx_v.at[pl.ds(j * hp, h)]], rows_v.at[bf], gsem[bf]
            )

        def wait_gather(j, bf):
            pltpu.make_async_copy(
                table_hbm.at[idx_v.at[pl.ds(j * hp, h)]], rows_v.at[bf], gsem[bf]
            ).wait()

        def start_write(j, bf):
            pltpu.async_copy(rows_v.at[bf], out_hbm.at[batch0 + j], wsem[bf])

        def wait_write(j, bf):
            pltpu.make_async_copy(
                rows_v.at[bf], out_hbm.at[batch0 + j], wsem[bf]
            ).wait()

        for bf in range(NBUF):
            start_gather(bf, bf)

        def body(r, carry):
            j0 = r * NBUF
            for bf in range(NBUF):
                wait_gather(j0 + bf, bf)
                start_write(j0 + bf, bf)
            for bf in range(NBUF):
                wait_write(j0 + bf, bf)
                start_gather(j0 + NBUF + bf, bf)
            return carry

        lax.fori_loop(0, rounds - 1, body, 0)

        j0 = (rounds - 1) * NBUF
        for bf in range(NBUF):
            wait_gather(j0 + bf, bf)
            start_write(j0 + bf, bf)
        for bf in range(NBUF):
            wait_write(j0 + bf, bf)

    return k(idx2_flat, pe)


def kernel(timestamps, pe):
    b, h = timestamps.shape
    d = pe.shape[1]
    hp = (h + 7) // 8 * 8
    idx2 = jnp.pad(timestamps, ((0, 0), (0, hp - h)), mode="edge")
    return _sc_gather(idx2.reshape(-1), pe, b, h, hp, d)
